# Initial kernel scaffold; baseline (speedup 1.0000x reference)
import jax
import jax.numpy as jnp
from jax.experimental import pallas as pl


def kernel(x, W_qkv, b_qkv, W_proj, b_proj):
    raise NotImplementedError("write your pallas kernel here")


# speedup vs baseline: 1.3935x; 1.3935x over previous
import functools

import jax
import jax.numpy as jnp
from jax.experimental import pallas as pl
from jax.experimental.pallas import tpu as pltpu

NUM_HEADS = 12
DIM = 768
HEAD_DIM = DIM // NUM_HEADS
BQ = 512


def _body(x_full_ref, x_blk_ref, wq_ref, wk_ref, wv_ref,
          bq_ref, bk_ref, bv_ref, wp_ref, bp_ref,
          out_ref, k_scr, v_scr, *, scale):
    i = pl.program_id(0)
    h = pl.program_id(1)

    @pl.when(i == 0)
    def _():
        xf = x_full_ref[...]
        k_scr[h] = (jnp.dot(xf, wk_ref[0], preferred_element_type=jnp.float32)
                    + bk_ref[0])
        v_scr[h] = (jnp.dot(xf, wv_ref[0], preferred_element_type=jnp.float32)
                    + bv_ref[0])

    q = (jnp.dot(x_blk_ref[...], wq_ref[0], preferred_element_type=jnp.float32)
         + bq_ref[0]) * scale
    s = jax.lax.dot_general(q, k_scr[h], (((1,), (1,)), ((), ())),
                            preferred_element_type=jnp.float32)
    s = s - jnp.max(s, axis=-1, keepdims=True)
    p = jnp.exp(s)
    o = jnp.dot(p, v_scr[h], preferred_element_type=jnp.float32)
    o = o / jnp.sum(p, axis=-1, keepdims=True)
    contrib = jnp.dot(o, wp_ref[...], preferred_element_type=jnp.float32)

    @pl.when(h == 0)
    def _():
        out_ref[...] = contrib + bp_ref[...]

    @pl.when(h > 0)
    def _():
        out_ref[...] += contrib


@jax.jit
def kernel(x, W_qkv, b_qkv, W_proj, b_proj):
    B, N, C = x.shape
    H, D = NUM_HEADS, HEAD_DIM
    scale = D ** -0.5
    x2 = x.reshape(N, C)
    W = W_qkv.reshape(C, 3, H, D)
    Wq = W[:, 0].transpose(1, 0, 2)
    Wk = W[:, 1].transpose(1, 0, 2)
    Wv = W[:, 2].transpose(1, 0, 2)
    b3 = b_qkv.reshape(3, H, 1, D)
    bq, bk, bv = b3[0], b3[1], b3[2]
    bp = b_proj.reshape(1, C)

    nq = N // BQ
    out = pl.pallas_call(
        functools.partial(_body, scale=scale),
        grid=(nq, H),
        in_specs=[
            pl.BlockSpec((N, C), lambda i, h: (0, 0)),
            pl.BlockSpec((BQ, C), lambda i, h: (i, 0)),
            pl.BlockSpec((1, C, D), lambda i, h: (h, 0, 0)),
            pl.BlockSpec((1, C, D), lambda i, h: (h, 0, 0)),
            pl.BlockSpec((1, C, D), lambda i, h: (h, 0, 0)),
            pl.BlockSpec((1, 1, D), lambda i, h: (h, 0, 0)),
            pl.BlockSpec((1, 1, D), lambda i, h: (h, 0, 0)),
            pl.BlockSpec((1, 1, D), lambda i, h: (h, 0, 0)),
            pl.BlockSpec((D, C), lambda i, h: (h, 0)),
            pl.BlockSpec((1, C), lambda i, h: (0, 0)),
        ],
        out_specs=pl.BlockSpec((BQ, C), lambda i, h: (i, 0)),
        out_shape=jax.ShapeDtypeStruct((N, C), jnp.float32),
        scratch_shapes=[
            pltpu.VMEM((H, N, D), jnp.float32),
            pltpu.VMEM((H, N, D), jnp.float32),
        ],
        compiler_params=pltpu.CompilerParams(
            dimension_semantics=("arbitrary", "arbitrary"),
        ),
    )(x2, x2, Wq, Wk, Wv, bq, bk, bv, W_proj, bp)
    return out.reshape(B, N, C)
